# Initial kernel scaffold; baseline (speedup 1.0000x reference)
import jax
import jax.numpy as jnp
from jax.experimental import pallas as pl


def kernel(text, words_per_sentence, emb_table, W_h, b_h, W_fc, b_fc):
    raise NotImplementedError("write your pallas kernel here")


# speedup vs baseline: 14.3728x; 14.3728x over previous
import jax
import jax.numpy as jnp
from jax import lax
from jax.experimental import pallas as pl
from jax.experimental.pallas import tpu as pltpu
import jax.experimental.pallas.tpu_sc as plsc

NC, NS, LANES = 2, 16, 16
NW = NC * NS

B, PADLEN, EMB = 4096, 200, 128
HIDDEN, NCLS = 512, 1000
BPW = B // NW
HALF = PADLEN // 2
NCHUNK = BPW * 2
NBUF = 4
ECH = EMB // LANES


def _reduce_chunk(buf_ref):
  def body(l, acc):
    return tuple(acc[j] + buf_ref[l, pl.ds(j * LANES, LANES)]
                 for j in range(ECH))
  init = tuple(jnp.zeros((LANES,), jnp.float32) for _ in range(ECH))
  return lax.fori_loop(0, HALF, body, init, unroll=4)


def _pool_body(text_hbm, table_hbm, out_hbm, idx_v, bufs, acc_v, sems):
  wid = lax.axis_index("s") * NC + lax.axis_index("c")
  base = wid * BPW

  pltpu.sync_copy(text_hbm.at[wid], idx_v)

  for b in range(NBUF):
    pltpu.async_copy(table_hbm.at[idx_v.at[b // 2, b % 2]], bufs[b], sems[b])

  def outer(i, carry):
    c0 = i * NBUF
    for k in range(NBUF):
      c = c0 + k
      pltpu.make_async_copy(table_hbm.at[idx_v.at[0, 0]], bufs[k],
                            sems[k]).wait()
      accs = _reduce_chunk(bufs[k])
      rr = c // 2
      if k % 2 == 0:
        for j in range(ECH):
          acc_v[rr, pl.ds(j * LANES, LANES)] = accs[j]
      else:
        for j in range(ECH):
          plsc.addupdate(acc_v.at[rr, pl.ds(j * LANES, LANES)], accs[j])
      nc = c + NBUF
      @pl.when(nc < NCHUNK)
      def _():
        pltpu.async_copy(table_hbm.at[idx_v.at[nc // 2, k % 2]], bufs[k],
                         sems[k])
    return carry

  lax.fori_loop(0, NCHUNK // NBUF, outer, 0)

  pltpu.sync_copy(acc_v, out_hbm.at[pl.ds(base, BPW)])


def _pool(text4, emb_table):
  mesh = plsc.VectorSubcoreMesh(core_axis_name="c", subcore_axis_name="s",
                                num_cores=NC, num_subcores=NS)
  kern = pl.kernel(
      _pool_body,
      out_type=jax.ShapeDtypeStruct((B, EMB), jnp.float32),
      mesh=mesh,
      scratch_types=[
          pltpu.VMEM((BPW, 2, HALF), jnp.int32),
          [pltpu.VMEM((HALF, EMB), jnp.float32) for _ in range(NBUF)],
          pltpu.VMEM((BPW, EMB), jnp.float32),
          [pltpu.SemaphoreType.DMA for _ in range(NBUF)],
      ],
  )
  return kern(text4, emb_table)


BB = 512
CPAD = 1024


def _mlp_body(x_ref, wh_ref, bh_ref, wfc_ref, bfc_ref, o_ref):
  x = x_ref[...] * jnp.float32(1.0 / PADLEN)
  h = jnp.dot(x, wh_ref[...], preferred_element_type=jnp.float32) + bh_ref[...]
  o_ref[...] = (jnp.dot(h, wfc_ref[...], preferred_element_type=jnp.float32)
                + bfc_ref[...])


def _mlp(sums, W_h, b_h, W_fc_p, b_fc_p):
  return pl.pallas_call(
      _mlp_body,
      grid=(B // BB,),
      in_specs=[
          pl.BlockSpec((BB, EMB), lambda i: (i, 0)),
          pl.BlockSpec((EMB, HIDDEN), lambda i: (0, 0)),
          pl.BlockSpec((1, HIDDEN), lambda i: (0, 0)),
          pl.BlockSpec((HIDDEN, CPAD), lambda i: (0, 0)),
          pl.BlockSpec((1, CPAD), lambda i: (0, 0)),
      ],
      out_specs=pl.BlockSpec((BB, CPAD), lambda i: (i, 0)),
      out_shape=jax.ShapeDtypeStruct((B, CPAD), jnp.float32),
  )(sums, W_h, b_h, W_fc_p, b_fc_p)


@jax.jit
def kernel(text, words_per_sentence, emb_table, W_h, b_h, W_fc, b_fc):
  del words_per_sentence
  text4 = text.astype(jnp.int32).reshape(NW, BPW, 2, HALF)
  sums = _pool(text4, emb_table)
  W_fc_p = jnp.pad(W_fc, ((0, 0), (0, CPAD - NCLS)))
  b_fc_p = jnp.pad(b_fc, (0, CPAD - NCLS)).reshape(1, CPAD)
  out = _mlp(sums, W_h, b_h.reshape(1, HIDDEN), W_fc_p, b_fc_p)
  return out[:, :NCLS]
